# Initial kernel scaffold; baseline (speedup 1.0000x reference)
import jax
import jax.numpy as jnp
from jax.experimental import pallas as pl


def kernel(x, W_proj, b_proj, W_ih, W_hh, b_ih, b_hh, W_sp, b_sp, W_router, b_router, W_out, b_out, prior_logit, queries):
    raise NotImplementedError("write your pallas kernel here")


# speedup vs baseline: 16.4986x; 16.4986x over previous
import jax
import jax.numpy as jnp
from jax.experimental import pallas as pl

_B, _N, _T = 4, 20000, 4
_D, _E, _M = 64, 32, 3
_TILE = 2048
_NP = 20480
_NT = _NP // _TILE
_K = 1000
_BITER = 45

_CT11 = (((1,), (1,)), ((), ()))


def _k1_body(s2_ref, wih_ref, wproj_ref, bproj_ref, bih_ref, bhh_ref,
             whh_ref, wsp_ref, bsp_ref, q_ref, wout_ref,
             scores_ref, hw_ref, feat_ref):
    nt = pl.program_id(1)
    f32 = jnp.float32

    wih = wih_ref[...]
    whh = whh_ref[...]
    wrow = wproj_ref[...].reshape(1, _D)
    brow = bproj_ref[...].reshape(1, _D)
    bih = bih_ref[...].reshape(1, 3 * _D)
    bhh = bhh_ref[...].reshape(1, 3 * _D)

    def gates(gi, gh, h):
        r = jax.nn.sigmoid(gi[:, :_D] + gh[:, :_D])
        z = jax.nn.sigmoid(gi[:, _D:2 * _D] + gh[:, _D:2 * _D])
        n = jnp.tanh(gi[:, 2 * _D:] + r * gh[:, 2 * _D:])
        return (1.0 - z) * n + z * h

    def gi_t(t):
        xt = s2_ref[:, t:t + 1] * wrow + brow
        return jax.lax.dot_general(xt, wih, _CT11,
                                   preferred_element_type=f32) + bih

    h = gates(gi_t(0), jnp.broadcast_to(bhh, (_TILE, 3 * _D)),
              jnp.zeros((_TILE, _D), f32))
    for t in range(1, _T):
        gh = jax.lax.dot_general(h, whh, _CT11,
                                 preferred_element_type=f32) + bhh
        h = gates(gi_t(t), gh, h)

    row = jax.lax.broadcasted_iota(jnp.int32, (_TILE, 1), 0)
    valid = (nt * _TILE + row) < _N

    hm = jnp.where(valid, h, 0.0)
    ssum = jnp.sum(hm, axis=0)
    ssq = jnp.sum(hm * hm, axis=0)

    @pl.when(nt == 0)
    def _():
        feat_ref[0, 0, :] = ssum
        feat_ref[0, 1, :] = ssq

    @pl.when(nt > 0)
    def _():
        feat_ref[0, 0, :] += ssum
        feat_ref[0, 1, :] += ssq

    hw = jax.lax.dot_general(wout_ref[...], h, _CT11,
                             preferred_element_type=f32)
    lane = jax.lax.broadcasted_iota(jnp.int32, (1, _TILE), 1)
    validT = (nt * _TILE + lane) < _N
    hw_ref[0, 0] = jnp.where(validT, hw, 0.0)

    for s in range(_M):
        a = jax.lax.dot_general(h, wsp_ref[s], _CT11,
                                preferred_element_type=f32) \
            + bsp_ref[s][None, :]
        nrm = jnp.maximum(jnp.sqrt(jnp.sum(a * a, axis=1, keepdims=True)),
                          1e-12)
        hs = a / nrm
        q = q_ref[s]
        qn = q / jnp.maximum(jnp.sqrt(jnp.sum(q * q, axis=1, keepdims=True)),
                             1e-12)
        sc = jax.lax.dot_general(hs, qn, _CT11,
                                 preferred_element_type=f32)
        scores_ref[0, s] = jnp.where(validT, sc.T, -2.0)


def _k2_body(scores_ref, hw_ref, feat_ref, wrt_ref, brt_ref, prior_ref,
             bout_ref, out_ref, probs_ref):
    f32 = jnp.float32
    sc = scores_ref[0]

    lo0 = jnp.full((_M, _E, 1), -1.001, f32)
    hi0 = jnp.full((_M, _E, 1), 1.001, f32)

    def bis(_, carry):
        lo, hi = carry
        mid = 0.5 * (lo + hi)
        cnt = jnp.sum((sc >= mid).astype(f32), axis=2, keepdims=True)
        ge = cnt >= float(_K)
        return jnp.where(ge, mid, lo), jnp.where(ge, hi, mid)

    lo, _hi = jax.lax.fori_loop(0, _BITER, bis, (lo0, hi0))

    mask = sc >= lo
    mx = jnp.max(sc, axis=2, keepdims=True)
    ex = jnp.where(mask, jnp.exp((sc - mx) * (1.0 / 0.7)), 0.0)
    w = ex / jnp.sum(ex, axis=2, keepdims=True)

    hw = hw_ref[0]
    g = jnp.sum(w * hw[None], axis=2)

    ssum = feat_ref[0, 0, :]
    ssq = feat_ref[0, 1, :]
    mean = ssum * (1.0 / _N)
    var = jnp.maximum((ssq - _N * mean * mean) * (1.0 / (_N - 1)), 0.0)
    feat = jnp.concatenate([mean, jnp.sqrt(var)])[None, :]
    logits = jnp.dot(feat, wrt_ref[...], preferred_element_type=f32) \
        + brt_ref[...]
    lmx = jnp.max(logits, axis=1, keepdims=True)
    el = jnp.exp(logits - lmx)
    probs = el / jnp.sum(el, axis=1, keepdims=True)
    probs_ref[0] = probs

    ev = jnp.dot(probs, g, preferred_element_type=f32)
    t = jnp.sum(w * ev[0][None, :, None], axis=1)
    outrow = jnp.dot(probs, t, preferred_element_type=f32)

    alpha = jax.nn.sigmoid(prior_ref[0, 0])
    out_ref[0] = (1.0 - alpha) * outrow + bout_ref[0, 0]


@jax.jit
def kernel(x, W_proj, b_proj, W_ih, W_hh, b_ih, b_hh, W_sp, b_sp,
           W_router, b_router, W_out, b_out, prior_logit, queries):
    f32 = jnp.float32
    s2 = jnp.transpose(x, (0, 2, 1, 3)).reshape(_B, _N, _T)
    s2 = jnp.pad(s2, ((0, 0), (0, _NP - _N), (0, 0))).reshape(_B * _NP, _T)

    bproj2 = b_proj.reshape(_D, 1)
    bih2 = b_ih.reshape(3 * _D, 1)
    bhh2 = b_hh.reshape(3 * _D, 1)
    wout2 = W_out.reshape(1, _D)
    wrt = W_router.T
    brt = b_router.reshape(1, _M)
    prior = prior_logit.reshape(1, 1)
    bout2 = b_out.reshape(1, 1)

    scores, hw, feat = pl.pallas_call(
        _k1_body,
        grid=(_B, _NT),
        in_specs=[
            pl.BlockSpec((_TILE, _T), lambda b, nt: (b * _NT + nt, 0)),
            pl.BlockSpec((3 * _D, _D), lambda b, nt: (0, 0)),
            pl.BlockSpec((_D, 1), lambda b, nt: (0, 0)),
            pl.BlockSpec((_D, 1), lambda b, nt: (0, 0)),
            pl.BlockSpec((3 * _D, 1), lambda b, nt: (0, 0)),
            pl.BlockSpec((3 * _D, 1), lambda b, nt: (0, 0)),
            pl.BlockSpec((3 * _D, _D), lambda b, nt: (0, 0)),
            pl.BlockSpec((_M, _D, _D), lambda b, nt: (0, 0, 0)),
            pl.BlockSpec((_M, _D), lambda b, nt: (0, 0)),
            pl.BlockSpec((_M, _E, _D), lambda b, nt: (0, 0, 0)),
            pl.BlockSpec((1, _D), lambda b, nt: (0, 0)),
        ],
        out_specs=[
            pl.BlockSpec((1, _M, _E, _TILE), lambda b, nt: (b, 0, 0, nt)),
            pl.BlockSpec((1, 1, 1, _TILE), lambda b, nt: (b, nt, 0, 0)),
            pl.BlockSpec((1, 2, _D), lambda b, nt: (b, 0, 0)),
        ],
        out_shape=[
            jax.ShapeDtypeStruct((_B, _M, _E, _NP), f32),
            jax.ShapeDtypeStruct((_B, _NT, 1, _TILE), f32),
            jax.ShapeDtypeStruct((_B, 2, _D), f32),
        ],
    )(s2, W_ih, W_proj, bproj2, bih2, bhh2, W_hh, W_sp, b_sp, queries, wout2)
    hw = hw.reshape(_B, 1, _NP)

    outp, probs = pl.pallas_call(
        _k2_body,
        grid=(_B,),
        in_specs=[
            pl.BlockSpec((1, _M, _E, _NP), lambda b: (b, 0, 0, 0)),
            pl.BlockSpec((1, 1, _NP), lambda b: (b, 0, 0)),
            pl.BlockSpec((1, 2, _D), lambda b: (b, 0, 0)),
            pl.BlockSpec((2 * _D, _M), lambda b: (0, 0)),
            pl.BlockSpec((1, _M), lambda b: (0, 0)),
            pl.BlockSpec((1, 1), lambda b: (0, 0)),
            pl.BlockSpec((1, 1), lambda b: (0, 0)),
        ],
        out_specs=[
            pl.BlockSpec((1, 1, _NP), lambda b: (b, 0, 0)),
            pl.BlockSpec((1, 1, _M), lambda b: (b, 0, 0)),
        ],
        out_shape=[
            jax.ShapeDtypeStruct((_B, 1, _NP), f32),
            jax.ShapeDtypeStruct((_B, 1, _M), f32),
        ],
    )(scores, hw, feat, wrt, brt, prior, bout2)

    return outp[:, 0, :_N], probs[:, 0, :]


# speedup vs baseline: 18.0393x; 1.0934x over previous
import jax
import jax.numpy as jnp
from jax.experimental import pallas as pl

_B, _N, _T = 4, 20000, 4
_D, _E, _M = 64, 32, 3
_TILE = 2048
_NP = 20480
_NT = _NP // _TILE
_K = 1000
_BITER = 31

import numpy as _np


def _mono_const(v):
    x = int(_np.float32(v).view(_np.int32))
    return x ^ ((x >> 31) & 0x7FFFFFFF)


_LOI = _mono_const(-1.001)
_HII = _mono_const(1.001)

_CT11 = (((1,), (1,)), ((), ()))


def _k1_body(s2_ref, wih_ref, wproj_ref, bproj_ref, bih_ref, bhh_ref,
             whh_ref, wsp_ref, bsp_ref, q_ref, wout_ref,
             scores_ref, hw_ref, feat_ref):
    nt = pl.program_id(1)
    f32 = jnp.float32

    wih = wih_ref[...]
    whh = whh_ref[...]
    wrow = wproj_ref[...].reshape(1, _D)
    brow = bproj_ref[...].reshape(1, _D)
    bih = bih_ref[...].reshape(1, 3 * _D)
    bhh = bhh_ref[...].reshape(1, 3 * _D)

    def gates(gi, gh, h):
        r = jax.nn.sigmoid(gi[:, :_D] + gh[:, :_D])
        z = jax.nn.sigmoid(gi[:, _D:2 * _D] + gh[:, _D:2 * _D])
        n = jnp.tanh(gi[:, 2 * _D:] + r * gh[:, 2 * _D:])
        return (1.0 - z) * n + z * h

    def gi_t(t):
        xt = s2_ref[:, t:t + 1] * wrow + brow
        return jax.lax.dot_general(xt, wih, _CT11,
                                   preferred_element_type=f32) + bih

    h = gates(gi_t(0), jnp.broadcast_to(bhh, (_TILE, 3 * _D)),
              jnp.zeros((_TILE, _D), f32))
    for t in range(1, _T):
        gh = jax.lax.dot_general(h, whh, _CT11,
                                 preferred_element_type=f32) + bhh
        h = gates(gi_t(t), gh, h)

    row = jax.lax.broadcasted_iota(jnp.int32, (_TILE, 1), 0)
    valid = (nt * _TILE + row) < _N

    hm = jnp.where(valid, h, 0.0)
    ssum = jnp.sum(hm, axis=0)
    ssq = jnp.sum(hm * hm, axis=0)

    @pl.when(nt == 0)
    def _():
        feat_ref[0, 0, :] = ssum
        feat_ref[0, 1, :] = ssq

    @pl.when(nt > 0)
    def _():
        feat_ref[0, 0, :] += ssum
        feat_ref[0, 1, :] += ssq

    hw = jax.lax.dot_general(wout_ref[...], h, _CT11,
                             preferred_element_type=f32)
    lane = jax.lax.broadcasted_iota(jnp.int32, (1, _TILE), 1)
    validT = (nt * _TILE + lane) < _N
    hw_ref[0, 0] = jnp.where(validT, hw, 0.0)

    for s in range(_M):
        a = jax.lax.dot_general(h, wsp_ref[s], _CT11,
                                preferred_element_type=f32) \
            + bsp_ref[s][None, :]
        nrm = jnp.maximum(jnp.sqrt(jnp.sum(a * a, axis=1, keepdims=True)),
                          1e-12)
        hs = a / nrm
        q = q_ref[s]
        qn = q / jnp.maximum(jnp.sqrt(jnp.sum(q * q, axis=1, keepdims=True)),
                             1e-12)
        sc = jax.lax.dot_general(hs, qn, _CT11,
                                 preferred_element_type=f32)
        scores_ref[0, s] = jnp.where(validT, sc.T, -2.0)


def _monotone_i32(x_i32):
    return x_i32 ^ ((x_i32 >> 31) & jnp.int32(0x7FFFFFFF))


def _k2_body(scores_ref, hw_ref, feat_ref, wrt_ref, brt_ref, prior_ref,
             bout_ref, out_ref, probs_ref):
    f32 = jnp.float32
    i32 = jnp.int32
    R = _M * _E
    sc = scores_ref[0].reshape(R, _NP)
    sci = _monotone_i32(jax.lax.bitcast_convert_type(sc, i32))
    ones = jnp.ones((1, _NP), f32)

    lo0 = jnp.full((R, 1), _LOI, i32)
    hi0 = jnp.full((R, 1), _HII, i32)

    def bis(_, carry):
        lo, hi = carry
        mid = (lo + hi) >> 1
        m = (sci >= mid).astype(f32)
        cnt = jax.lax.dot_general(m, ones, _CT11,
                                  preferred_element_type=f32)
        ge = cnt >= float(_K)
        return jnp.where(ge, mid, lo), jnp.where(ge, hi, mid)

    lo, _hi = jax.lax.fori_loop(0, _BITER, bis, (lo0, hi0))

    mask = sci >= lo
    mx = jnp.max(sc, axis=1, keepdims=True)
    ex = jnp.where(mask, jnp.exp((sc - mx) * (1.0 / 0.7)), 0.0)
    denom = jax.lax.dot_general(ex, ones, _CT11,
                                preferred_element_type=f32)
    hw = hw_ref[0]
    gnum = jax.lax.dot_general(ex, hw, _CT11,
                               preferred_element_type=f32)
    g = (gnum / denom).reshape(_M, _E)

    ssum = feat_ref[0, 0, :]
    ssq = feat_ref[0, 1, :]
    mean = ssum * (1.0 / _N)
    var = jnp.maximum((ssq - _N * mean * mean) * (1.0 / (_N - 1)), 0.0)
    feat = jnp.concatenate([mean, jnp.sqrt(var)])[None, :]
    logits = jnp.dot(feat, wrt_ref[...], preferred_element_type=f32) \
        + brt_ref[...]
    lmx = jnp.max(logits, axis=1, keepdims=True)
    el = jnp.exp(logits - lmx)
    probs = el / jnp.sum(el, axis=1, keepdims=True)
    probs_ref[0] = probs

    ev = jnp.dot(probs, g, preferred_element_type=f32)
    coeff = (probs.reshape(_M, 1) * ev) / denom.reshape(_M, _E)
    t3 = jax.lax.dot_general(coeff, ex.reshape(_M, _E, _NP),
                             (((1,), (1,)), ((0,), (0,))),
                             preferred_element_type=f32)
    outrow = jnp.sum(t3, axis=0, keepdims=True)

    alpha = jax.nn.sigmoid(prior_ref[0, 0])
    out_ref[0] = (1.0 - alpha) * outrow + bout_ref[0, 0]


@jax.jit
def kernel(x, W_proj, b_proj, W_ih, W_hh, b_ih, b_hh, W_sp, b_sp,
           W_router, b_router, W_out, b_out, prior_logit, queries):
    f32 = jnp.float32
    s2 = jnp.transpose(x, (0, 2, 1, 3)).reshape(_B, _N, _T)
    s2 = jnp.pad(s2, ((0, 0), (0, _NP - _N), (0, 0))).reshape(_B * _NP, _T)

    bproj2 = b_proj.reshape(_D, 1)
    bih2 = b_ih.reshape(3 * _D, 1)
    bhh2 = b_hh.reshape(3 * _D, 1)
    wout2 = W_out.reshape(1, _D)
    wrt = W_router.T
    brt = b_router.reshape(1, _M)
    prior = prior_logit.reshape(1, 1)
    bout2 = b_out.reshape(1, 1)

    scores, hw, feat = pl.pallas_call(
        _k1_body,
        grid=(_B, _NT),
        in_specs=[
            pl.BlockSpec((_TILE, _T), lambda b, nt: (b * _NT + nt, 0)),
            pl.BlockSpec((3 * _D, _D), lambda b, nt: (0, 0)),
            pl.BlockSpec((_D, 1), lambda b, nt: (0, 0)),
            pl.BlockSpec((_D, 1), lambda b, nt: (0, 0)),
            pl.BlockSpec((3 * _D, 1), lambda b, nt: (0, 0)),
            pl.BlockSpec((3 * _D, 1), lambda b, nt: (0, 0)),
            pl.BlockSpec((3 * _D, _D), lambda b, nt: (0, 0)),
            pl.BlockSpec((_M, _D, _D), lambda b, nt: (0, 0, 0)),
            pl.BlockSpec((_M, _D), lambda b, nt: (0, 0)),
            pl.BlockSpec((_M, _E, _D), lambda b, nt: (0, 0, 0)),
            pl.BlockSpec((1, _D), lambda b, nt: (0, 0)),
        ],
        out_specs=[
            pl.BlockSpec((1, _M, _E, _TILE), lambda b, nt: (b, 0, 0, nt)),
            pl.BlockSpec((1, 1, 1, _TILE), lambda b, nt: (b, nt, 0, 0)),
            pl.BlockSpec((1, 2, _D), lambda b, nt: (b, 0, 0)),
        ],
        out_shape=[
            jax.ShapeDtypeStruct((_B, _M, _E, _NP), f32),
            jax.ShapeDtypeStruct((_B, _NT, 1, _TILE), f32),
            jax.ShapeDtypeStruct((_B, 2, _D), f32),
        ],
    )(s2, W_ih, W_proj, bproj2, bih2, bhh2, W_hh, W_sp, b_sp, queries, wout2)
    hw = hw.reshape(_B, 1, _NP)

    outp, probs = pl.pallas_call(
        _k2_body,
        grid=(_B,),
        in_specs=[
            pl.BlockSpec((1, _M, _E, _NP), lambda b: (b, 0, 0, 0)),
            pl.BlockSpec((1, 1, _NP), lambda b: (b, 0, 0)),
            pl.BlockSpec((1, 2, _D), lambda b: (b, 0, 0)),
            pl.BlockSpec((2 * _D, _M), lambda b: (0, 0)),
            pl.BlockSpec((1, _M), lambda b: (0, 0)),
            pl.BlockSpec((1, 1), lambda b: (0, 0)),
            pl.BlockSpec((1, 1), lambda b: (0, 0)),
        ],
        out_specs=[
            pl.BlockSpec((1, 1, _NP), lambda b: (b, 0, 0)),
            pl.BlockSpec((1, 1, _M), lambda b: (b, 0, 0)),
        ],
        out_shape=[
            jax.ShapeDtypeStruct((_B, 1, _NP), f32),
            jax.ShapeDtypeStruct((_B, 1, _M), f32),
        ],
    )(scores, hw, feat, wrt, brt, prior, bout2)

    return outp[:, 0, :_N], probs[:, 0, :]


# speedup vs baseline: 18.1141x; 1.0041x over previous
import jax
import jax.numpy as jnp
from jax.experimental import pallas as pl

_B, _N, _T = 4, 20000, 4
_D, _E, _M = 64, 32, 3
_TILE = 4096
_NP = 20480
_NT = _NP // _TILE
_K = 1000
_BITER = 31

import numpy as _np


def _mono_const(v):
    x = int(_np.float32(v).view(_np.int32))
    return x ^ ((x >> 31) & 0x7FFFFFFF)


_LOI = _mono_const(-1.001)
_HII = _mono_const(1.001)

_CT11 = (((1,), (1,)), ((), ()))


def _k1_body(s2_ref, wih_ref, wproj_ref, bproj_ref, bih_ref, bhh_ref,
             whh_ref, wsp_ref, bspc_ref, q_ref, wout_ref,
             scores_ref, hw_ref, feat_ref):
    nt = pl.program_id(1)
    f32 = jnp.float32

    wih = wih_ref[...]
    whh = whh_ref[...]
    wrow = wproj_ref[...].reshape(1, _D)
    brow = bproj_ref[...].reshape(1, _D)
    bih = bih_ref[...].reshape(1, 3 * _D)
    bhh = bhh_ref[...].reshape(1, 3 * _D)

    def gates(gi, gh, h):
        r = jax.nn.sigmoid(gi[:, :_D] + gh[:, :_D])
        z = jax.nn.sigmoid(gi[:, _D:2 * _D] + gh[:, _D:2 * _D])
        n = jnp.tanh(gi[:, 2 * _D:] + r * gh[:, 2 * _D:])
        return (1.0 - z) * n + z * h

    def gi_t(t):
        xt = s2_ref[:, t:t + 1] * wrow + brow
        return jax.lax.dot_general(xt, wih, _CT11,
                                   preferred_element_type=f32) + bih

    h = gates(gi_t(0), jnp.broadcast_to(bhh, (_TILE, 3 * _D)),
              jnp.zeros((_TILE, _D), f32))
    for t in range(1, _T):
        gh = jax.lax.dot_general(h, whh, _CT11,
                                 preferred_element_type=f32) + bhh
        h = gates(gi_t(t), gh, h)

    row = jax.lax.broadcasted_iota(jnp.int32, (_TILE, 1), 0)
    valid = (nt * _TILE + row) < _N

    hm = jnp.where(valid, h, 0.0)
    ssum = jnp.sum(hm, axis=0)
    ssq = jnp.sum(hm * hm, axis=0)

    @pl.when(nt == 0)
    def _():
        feat_ref[0, 0, :] = ssum
        feat_ref[0, 1, :] = ssq

    @pl.when(nt > 0)
    def _():
        feat_ref[0, 0, :] += ssum
        feat_ref[0, 1, :] += ssq

    hw = jax.lax.dot_general(wout_ref[...], h, _CT11,
                             preferred_element_type=f32)
    lane = jax.lax.broadcasted_iota(jnp.int32, (1, _TILE), 1)
    validT = (nt * _TILE + lane) < _N
    hw_ref[0, 0] = jnp.where(validT, hw, 0.0)

    wspc = wsp_ref[...].reshape(_M * _D, _D)
    a_all = jax.lax.dot_general(h, wspc, _CT11,
                                preferred_element_type=f32) \
        + bspc_ref[...]
    aa = a_all * a_all
    rid = jax.lax.broadcasted_iota(jnp.int32, (_M, _M * _D), 1)
    sid = jax.lax.broadcasted_iota(jnp.int32, (_M, _M * _D), 0)
    sel = ((rid // _D) == sid).astype(f32)
    nrm2 = jax.lax.dot_general(aa, sel, _CT11,
                               preferred_element_type=f32)
    nrm = jnp.maximum(jnp.sqrt(nrm2), 1e-12)
    for s in range(_M):
        hs = a_all[:, s * _D:(s + 1) * _D] / nrm[:, s:s + 1]
        q = q_ref[s]
        qn = q / jnp.maximum(jnp.sqrt(jnp.sum(q * q, axis=1, keepdims=True)),
                             1e-12)
        sc = jax.lax.dot_general(hs, qn, _CT11,
                                 preferred_element_type=f32)
        scores_ref[0, s] = jnp.where(validT, sc.T, -2.0)


def _monotone_i32(x_i32):
    return x_i32 ^ ((x_i32 >> 31) & jnp.int32(0x7FFFFFFF))


def _k2_body(scores_ref, hw_ref, feat_ref, wrt_ref, brt_ref, prior_ref,
             bout_ref, out_ref, probs_ref):
    f32 = jnp.float32
    i32 = jnp.int32
    R = _M * _E
    sc = scores_ref[0].reshape(R, _NP)
    sci = _monotone_i32(jax.lax.bitcast_convert_type(sc, i32))
    ones = jnp.ones((1, _NP), f32)

    lo0 = jnp.full((R, 1), _LOI, i32)
    hi0 = jnp.full((R, 1), _HII, i32)

    def bis(_, carry):
        lo, hi = carry
        mid = (lo + hi) >> 1
        m = (sci >= mid).astype(f32)
        cnt = jax.lax.dot_general(m, ones, _CT11,
                                  preferred_element_type=f32)
        ge = cnt >= float(_K)
        return jnp.where(ge, mid, lo), jnp.where(ge, hi, mid)

    lo, _hi = jax.lax.fori_loop(0, _BITER, bis, (lo0, hi0))

    mask = sci >= lo
    mx = jnp.max(sc, axis=1, keepdims=True)
    ex = jnp.where(mask, jnp.exp((sc - mx) * (1.0 / 0.7)), 0.0)
    denom = jax.lax.dot_general(ex, ones, _CT11,
                                preferred_element_type=f32)
    hw = hw_ref[0]
    gnum = jax.lax.dot_general(ex, hw, _CT11,
                               preferred_element_type=f32)
    g = (gnum / denom).reshape(_M, _E)

    ssum = feat_ref[0, 0, :]
    ssq = feat_ref[0, 1, :]
    mean = ssum * (1.0 / _N)
    var = jnp.maximum((ssq - _N * mean * mean) * (1.0 / (_N - 1)), 0.0)
    feat = jnp.concatenate([mean, jnp.sqrt(var)])[None, :]
    logits = jnp.dot(feat, wrt_ref[...], preferred_element_type=f32) \
        + brt_ref[...]
    lmx = jnp.max(logits, axis=1, keepdims=True)
    el = jnp.exp(logits - lmx)
    probs = el / jnp.sum(el, axis=1, keepdims=True)
    probs_ref[0] = probs

    ev = jnp.dot(probs, g, preferred_element_type=f32)
    coeff = (probs.reshape(_M, 1) * ev) / denom.reshape(_M, _E)
    t3 = jax.lax.dot_general(coeff, ex.reshape(_M, _E, _NP),
                             (((1,), (1,)), ((0,), (0,))),
                             preferred_element_type=f32)
    outrow = jnp.sum(t3, axis=0, keepdims=True)

    alpha = jax.nn.sigmoid(prior_ref[0, 0])
    out_ref[0] = (1.0 - alpha) * outrow + bout_ref[0, 0]


@jax.jit
def kernel(x, W_proj, b_proj, W_ih, W_hh, b_ih, b_hh, W_sp, b_sp,
           W_router, b_router, W_out, b_out, prior_logit, queries):
    f32 = jnp.float32
    s2 = jnp.transpose(x, (0, 2, 1, 3)).reshape(_B, _N, _T)
    s2 = jnp.pad(s2, ((0, 0), (0, _NP - _N), (0, 0))).reshape(_B * _NP, _T)

    bproj2 = b_proj.reshape(_D, 1)
    bih2 = b_ih.reshape(3 * _D, 1)
    bhh2 = b_hh.reshape(3 * _D, 1)
    wout2 = W_out.reshape(1, _D)
    wrt = W_router.T
    brt = b_router.reshape(1, _M)
    prior = prior_logit.reshape(1, 1)
    bout2 = b_out.reshape(1, 1)

    scores, hw, feat = pl.pallas_call(
        _k1_body,
        grid=(_B, _NT),
        in_specs=[
            pl.BlockSpec((_TILE, _T), lambda b, nt: (b * _NT + nt, 0)),
            pl.BlockSpec((3 * _D, _D), lambda b, nt: (0, 0)),
            pl.BlockSpec((_D, 1), lambda b, nt: (0, 0)),
            pl.BlockSpec((_D, 1), lambda b, nt: (0, 0)),
            pl.BlockSpec((3 * _D, 1), lambda b, nt: (0, 0)),
            pl.BlockSpec((3 * _D, 1), lambda b, nt: (0, 0)),
            pl.BlockSpec((3 * _D, _D), lambda b, nt: (0, 0)),
            pl.BlockSpec((_M, _D, _D), lambda b, nt: (0, 0, 0)),
            pl.BlockSpec((1, _M * _D), lambda b, nt: (0, 0)),
            pl.BlockSpec((_M, _E, _D), lambda b, nt: (0, 0, 0)),
            pl.BlockSpec((1, _D), lambda b, nt: (0, 0)),
        ],
        out_specs=[
            pl.BlockSpec((1, _M, _E, _TILE), lambda b, nt: (b, 0, 0, nt)),
            pl.BlockSpec((1, 1, 1, _TILE), lambda b, nt: (b, nt, 0, 0)),
            pl.BlockSpec((1, 2, _D), lambda b, nt: (b, 0, 0)),
        ],
        out_shape=[
            jax.ShapeDtypeStruct((_B, _M, _E, _NP), f32),
            jax.ShapeDtypeStruct((_B, _NT, 1, _TILE), f32),
            jax.ShapeDtypeStruct((_B, 2, _D), f32),
        ],
    )(s2, W_ih, W_proj, bproj2, bih2, bhh2, W_hh, W_sp,
      b_sp.reshape(1, _M * _D), queries, wout2)
    hw = hw.reshape(_B, 1, _NP)

    outp, probs = pl.pallas_call(
        _k2_body,
        grid=(_B,),
        in_specs=[
            pl.BlockSpec((1, _M, _E, _NP), lambda b: (b, 0, 0, 0)),
            pl.BlockSpec((1, 1, _NP), lambda b: (b, 0, 0)),
            pl.BlockSpec((1, 2, _D), lambda b: (b, 0, 0)),
            pl.BlockSpec((2 * _D, _M), lambda b: (0, 0)),
            pl.BlockSpec((1, _M), lambda b: (0, 0)),
            pl.BlockSpec((1, 1), lambda b: (0, 0)),
            pl.BlockSpec((1, 1), lambda b: (0, 0)),
        ],
        out_specs=[
            pl.BlockSpec((1, 1, _NP), lambda b: (b, 0, 0)),
            pl.BlockSpec((1, 1, _M), lambda b: (b, 0, 0)),
        ],
        out_shape=[
            jax.ShapeDtypeStruct((_B, 1, _NP), f32),
            jax.ShapeDtypeStruct((_B, 1, _M), f32),
        ],
    )(scores, hw, feat, wrt, brt, prior, bout2)

    return outp[:, 0, :_N], probs[:, 0, :]


# speedup vs baseline: 18.1713x; 1.0032x over previous
import jax
import jax.numpy as jnp
import numpy as _np
from jax.experimental import pallas as pl
from jax.experimental.pallas import tpu as pltpu

_B, _N, _T = 4, 20000, 4
_D, _E, _M = 64, 32, 3
_TILE = 2048
_NP = 20480
_NT = _NP // _TILE
_K = 1000
_BITER = 31


def _mono_const(v):
    x = int(_np.float32(v).view(_np.int32))
    return x ^ ((x >> 31) & 0x7FFFFFFF)


_LOI = _mono_const(-1.001)
_HII = _mono_const(1.001)

_CT11 = (((1,), (1,)), ((), ()))


def _monotone_i32(x_i32):
    return x_i32 ^ ((x_i32 >> 31) & jnp.int32(0x7FFFFFFF))


def _body(s2_ref, wih_ref, wproj_ref, bproj_ref, bih_ref, bhh_ref,
          whh_ref, wsp_ref, bsp_ref, q_ref, wout_ref,
          wrt_ref, brt_ref, prior_ref, bout_ref,
          out_ref, probs_ref,
          sc_s, hw_s, feat_s):
    nt = pl.program_id(1)
    f32 = jnp.float32
    i32 = jnp.int32

    @pl.when(nt < _NT)
    def _tile_step():
        wih = wih_ref[...]
        whh = whh_ref[...]
        wrow = wproj_ref[...].reshape(1, _D)
        brow = bproj_ref[...].reshape(1, _D)
        bih = bih_ref[...].reshape(1, 3 * _D)
        bhh = bhh_ref[...].reshape(1, 3 * _D)

        def gates(gi, gh, h):
            r = jax.nn.sigmoid(gi[:, :_D] + gh[:, :_D])
            z = jax.nn.sigmoid(gi[:, _D:2 * _D] + gh[:, _D:2 * _D])
            n = jnp.tanh(gi[:, 2 * _D:] + r * gh[:, 2 * _D:])
            return (1.0 - z) * n + z * h

        def gi_t(t):
            xt = s2_ref[:, t:t + 1] * wrow + brow
            return jax.lax.dot_general(xt, wih, _CT11,
                                       preferred_element_type=f32) + bih

        h = gates(gi_t(0), jnp.broadcast_to(bhh, (_TILE, 3 * _D)),
                  jnp.zeros((_TILE, _D), f32))
        for t in range(1, _T):
            gh = jax.lax.dot_general(h, whh, _CT11,
                                     preferred_element_type=f32) + bhh
            h = gates(gi_t(t), gh, h)

        row = jax.lax.broadcasted_iota(i32, (_TILE, 1), 0)
        valid = (nt * _TILE + row) < _N

        hm = jnp.where(valid, h, 0.0)
        ssum = jnp.sum(hm, axis=0)
        ssq = jnp.sum(hm * hm, axis=0)

        @pl.when(nt == 0)
        def _():
            feat_s[0, :] = ssum
            feat_s[1, :] = ssq

        @pl.when(nt > 0)
        def _():
            feat_s[0, :] += ssum
            feat_s[1, :] += ssq

        hw = jax.lax.dot_general(wout_ref[...], h, _CT11,
                                 preferred_element_type=f32)
        lane = jax.lax.broadcasted_iota(i32, (1, _TILE), 1)
        validT = (nt * _TILE + lane) < _N
        hw_s[:, pl.ds(nt * _TILE, _TILE)] = jnp.where(validT, hw, 0.0)

        for s in range(_M):
            a = jax.lax.dot_general(h, wsp_ref[s], _CT11,
                                    preferred_element_type=f32) \
                + bsp_ref[s][None, :]
            nrm = jnp.maximum(
                jnp.sqrt(jnp.sum(a * a, axis=1, keepdims=True)), 1e-12)
            hs = a / nrm
            q = q_ref[s]
            qn = q / jnp.maximum(
                jnp.sqrt(jnp.sum(q * q, axis=1, keepdims=True)), 1e-12)
            sc = jax.lax.dot_general(hs, qn, _CT11,
                                     preferred_element_type=f32)
            sc_s[s, :, pl.ds(nt * _TILE, _TILE)] = jnp.where(validT, sc.T,
                                                             -2.0)

    @pl.when(nt == _NT)
    def _select_step():
        R = _M * _E
        sc = sc_s[...].reshape(R, _NP)
        sci = _monotone_i32(jax.lax.bitcast_convert_type(sc, i32))
        ones = jnp.ones((1, _NP), f32)

        lo0 = jnp.full((R, 1), _LOI, i32)
        hi0 = jnp.full((R, 1), _HII, i32)

        def bis(_, carry):
            lo, hi = carry
            mid = (lo + hi) >> 1
            m = (sci >= mid).astype(f32)
            cnt = jax.lax.dot_general(m, ones, _CT11,
                                      preferred_element_type=f32)
            ge = cnt >= float(_K)
            return jnp.where(ge, mid, lo), jnp.where(ge, hi, mid)

        lo, _hi = jax.lax.fori_loop(0, _BITER, bis, (lo0, hi0))

        mask = sci >= lo
        mx = jnp.max(sc, axis=1, keepdims=True)
        ex = jnp.where(mask, jnp.exp((sc - mx) * (1.0 / 0.7)), 0.0)
        denom = jax.lax.dot_general(ex, ones, _CT11,
                                    preferred_element_type=f32)
        hw = hw_s[...]
        gnum = jax.lax.dot_general(ex, hw, _CT11,
                                   preferred_element_type=f32)
        g = (gnum / denom).reshape(_M, _E)

        ssum = feat_s[0, :]
        ssq = feat_s[1, :]
        mean = ssum * (1.0 / _N)
        var = jnp.maximum((ssq - _N * mean * mean) * (1.0 / (_N - 1)), 0.0)
        feat = jnp.concatenate([mean, jnp.sqrt(var)])[None, :]
        logits = jnp.dot(feat, wrt_ref[...], preferred_element_type=f32) \
            + brt_ref[...]
        lmx = jnp.max(logits, axis=1, keepdims=True)
        el = jnp.exp(logits - lmx)
        probs = el / jnp.sum(el, axis=1, keepdims=True)
        probs_ref[0] = probs

        ev = jnp.dot(probs, g, preferred_element_type=f32)
        coeff = (probs.reshape(_M, 1) * ev) / denom.reshape(_M, _E)
        t3 = jax.lax.dot_general(coeff, ex.reshape(_M, _E, _NP),
                                 (((1,), (1,)), ((0,), (0,))),
                                 preferred_element_type=f32)
        outrow = jnp.sum(t3, axis=0, keepdims=True)

        alpha = jax.nn.sigmoid(prior_ref[0, 0])
        out_ref[0] = (1.0 - alpha) * outrow + bout_ref[0, 0]


@jax.jit
def kernel(x, W_proj, b_proj, W_ih, W_hh, b_ih, b_hh, W_sp, b_sp,
           W_router, b_router, W_out, b_out, prior_logit, queries):
    f32 = jnp.float32
    s2 = jnp.transpose(x, (0, 2, 1, 3)).reshape(_B, _N, _T)
    s2 = jnp.pad(s2, ((0, 0), (0, _NP - _N), (0, 0))).reshape(_B * _NP, _T)

    bproj2 = b_proj.reshape(_D, 1)
    bih2 = b_ih.reshape(3 * _D, 1)
    bhh2 = b_hh.reshape(3 * _D, 1)
    wout2 = W_out.reshape(1, _D)
    wrt = W_router.T
    brt = b_router.reshape(1, _M)
    prior = prior_logit.reshape(1, 1)
    bout2 = b_out.reshape(1, 1)

    fixed = lambda b, nt: (0, 0)
    fixed3 = lambda b, nt: (0, 0, 0)

    outp, probs = pl.pallas_call(
        _body,
        grid=(_B, _NT + 1),
        in_specs=[
            pl.BlockSpec((_TILE, _T),
                         lambda b, nt: (b * _NT + jnp.minimum(nt, _NT - 1),
                                        0)),
            pl.BlockSpec((3 * _D, _D), fixed),
            pl.BlockSpec((_D, 1), fixed),
            pl.BlockSpec((_D, 1), fixed),
            pl.BlockSpec((3 * _D, 1), fixed),
            pl.BlockSpec((3 * _D, 1), fixed),
            pl.BlockSpec((3 * _D, _D), fixed),
            pl.BlockSpec((_M, _D, _D), fixed3),
            pl.BlockSpec((_M, _D), fixed),
            pl.BlockSpec((_M, _E, _D), fixed3),
            pl.BlockSpec((1, _D), fixed),
            pl.BlockSpec((2 * _D, _M), fixed),
            pl.BlockSpec((1, _M), fixed),
            pl.BlockSpec((1, 1), fixed),
            pl.BlockSpec((1, 1), fixed),
        ],
        out_specs=[
            pl.BlockSpec((1, 1, _NP), lambda b, nt: (b, 0, 0)),
            pl.BlockSpec((1, 1, _M), lambda b, nt: (b, 0, 0)),
        ],
        out_shape=[
            jax.ShapeDtypeStruct((_B, 1, _NP), f32),
            jax.ShapeDtypeStruct((_B, 1, _M), f32),
        ],
        scratch_shapes=[
            pltpu.VMEM((_M, _E, _NP), f32),
            pltpu.VMEM((1, _NP), f32),
            pltpu.VMEM((2, _D), f32),
        ],
    )(s2, W_ih, W_proj, bproj2, bih2, bhh2, W_hh, W_sp, b_sp, queries, wout2,
      wrt, brt, prior, bout2)

    return outp[:, 0, :_N], probs[:, 0, :]
